# baseline (speedup 1.0000x reference)
import functools

import jax
import jax.numpy as jnp
import numpy as np
from jax import lax
from jax.experimental import pallas as pl
from jax.experimental.pallas import tpu as pltpu
from jax.experimental.pallas import tpu_sc as plsc

B, S, D = 1, 8192, 1024
H, DH = 16, 64
NH = 4
NB = 256
NKEY = NH * NB
CHUNK = 64
NS = NH * S
NC = NS // CHUNK


_SB = 512
_HB = 2048


def _stage_a1_body(hs_ref, wqk_ref, wv_ref, qkv_ref, qkt_ref):
  hs = hs_ref[...]
  qk = lax.dot_general(hs, wqk_ref[...], (((1,), (1,)), ((), ())),
                       preferred_element_type=jnp.float32)
  v = lax.dot_general(hs, wv_ref[...], (((1,), (1,)), ((), ())),
                      preferred_element_type=jnp.float32)
  qkt_ref[...] = qk.T
  for h in range(H):
    qkv_ref[:, h, 0:DH] = qk[:, h * DH:(h + 1) * DH]
    qkv_ref[:, h, DH:2 * DH] = v[:, h * DH:(h + 1) * DH]


def _stage_a1(hs, w_qk, w_v):
  return pl.pallas_call(
      _stage_a1_body,
      grid=(S // _SB,),
      in_specs=[
          pl.BlockSpec((_SB, D), lambda i: (i, 0)),
          pl.BlockSpec((H * DH, D), lambda i: (0, 0)),
          pl.BlockSpec((H * DH, D), lambda i: (0, 0)),
      ],
      out_specs=[
          pl.BlockSpec((_SB, H, 2 * DH), lambda i: (i, 0, 0)),
          pl.BlockSpec((H * DH, _SB), lambda i: (0, i)),
      ],
      out_shape=[
          jax.ShapeDtypeStruct((S, H, 2 * DH), jnp.float32),
          jax.ShapeDtypeStruct((H * DH, S), jnp.float32),
      ],
      compiler_params=pltpu.CompilerParams(
          dimension_semantics=("arbitrary",),
      ),
  )(hs, w_qk, w_v)


def _stage_a2_body(qkt_ref, rot_ref, bkt_ref):
  rt = lax.dot_general(rot_ref[...], qkt_ref[...], (((0,), (0,)), ((), ())),
                       preferred_element_type=jnp.float32)
  sub = lax.broadcasted_iota(jnp.int32, (NB // 2, _HB), 0)
  big = jnp.int32(1 << 20)
  for n in range(NH):
    rn = rt[n * (NB // 2):(n + 1) * (NB // 2), :]
    mx = jnp.max(rn, axis=0, keepdims=True)
    mn = jnp.min(rn, axis=0, keepdims=True)
    amx = jnp.min(jnp.where(rn == mx, sub, big), axis=0, keepdims=True)
    amn = jnp.min(jnp.where(rn == mn, sub, big), axis=0, keepdims=True)
    b = jnp.where(mx >= -mn, amx, NB // 2 + amn) + n * NB
    bkt_ref[0, n:n + 1, :] = b


def _stage_a2(qkt, rot2):
  return pl.pallas_call(
      _stage_a2_body,
      grid=(H, S // _HB),
      in_specs=[
          pl.BlockSpec((DH, _HB), lambda h, j: (h, j)),
          pl.BlockSpec((DH, NH * (NB // 2)), lambda h, j: (0, 0)),
      ],
      out_specs=pl.BlockSpec((1, NH, _HB), lambda h, j: (h, 0, j)),
      out_shape=jax.ShapeDtypeStruct((H, NH, S), jnp.int32),
      compiler_params=pltpu.CompilerParams(
          dimension_semantics=("arbitrary", "arbitrary"),
      ),
  )(qkt, rot2)


_NVEC = NS // 16


def _stage_b_body(bkt_hbm, rank_hbm, pos_hbm, bkt_v, hist_v, rank_v, pos_v):
  wid = lax.axis_index("s") * 2 + lax.axis_index("c")

  @pl.when(wid < H)
  def _():
    pltpu.sync_copy(bkt_hbm.at[wid], bkt_v)

    def zero_hist(i, _):
      hist_v[pl.ds(i * 16, 16)] = jnp.zeros((16,), jnp.int32)
      return 0
    lax.fori_loop(0, NKEY // 16, zero_hist, 0)

    def hist_pass(k, _):
      b = bkt_v[pl.ds(k * 16, 16)]
      cnt, last = plsc.scan_count(b)
      base = plsc.load_gather(hist_v, [b])
      plsc.store_scatter(hist_v, [b], base + cnt, mask=last)
      return 0
    lax.fori_loop(0, _NVEC, hist_pass, 0)

    def excl_scan(i, carry):
      v = hist_v[pl.ds(i * 16, 16)]
      c = plsc.cumsum(v)
      hist_v[pl.ds(i * 16, 16)] = carry + c - v
      return carry + jnp.sum(v)
    lax.fori_loop(0, NKEY // 16, excl_scan, jnp.int32(0))

    def rank_pass(k, _):
      b = bkt_v[pl.ds(k * 16, 16)]
      cnt, last = plsc.scan_count(b)
      base = plsc.load_gather(hist_v, [b])
      rank_v[pl.ds(k * 16, 16)] = base + cnt - 1
      plsc.store_scatter(hist_v, [b], base + cnt, mask=last)
      return 0
    lax.fori_loop(0, _NVEC, rank_pass, 0)

    pltpu.sync_copy(rank_v, rank_hbm.at[wid])

    def pos_pass(k, _):
      r = rank_v[pl.ds(k * 16, 16)]
      i_vec = k * 16 + lax.iota(jnp.int32, 16)
      plsc.store_scatter(pos_v, [r], i_vec & (S - 1))
      return 0
    lax.fori_loop(0, _NVEC, pos_pass, 0)
    pltpu.sync_copy(pos_v, pos_hbm.at[wid])


def _stage_b(buckets):
  mesh = plsc.VectorSubcoreMesh(core_axis_name="c", subcore_axis_name="s")
  f = pl.kernel(
      _stage_b_body,
      out_type=[jax.ShapeDtypeStruct((H, NS), jnp.int32),
                jax.ShapeDtypeStruct((H, NS), jnp.int32)],
      mesh=mesh,
      scratch_types=[
          pltpu.VMEM((NS,), jnp.int32),
          pltpu.VMEM((NKEY,), jnp.int32),
          pltpu.VMEM((NS,), jnp.int32),
          pltpu.VMEM((NS,), jnp.int32),
      ],
      compiler_params=pltpu.CompilerParams(needs_layout_passes=False),
  )
  return f(buckets.reshape(H, NS))


_CW = 128


def _stage_c_body(qkv_hbm, rank_hbm, qkvg_hbm, rows_v, idx_v, sem):
  wid = lax.axis_index("s") * 2 + lax.axis_index("c")
  h = wid // 2
  half = wid % 2

  for n2 in range(2):
    def window(w, _, n2=n2):
      n = half * 2 + n2
      s0 = w * _CW
      pltpu.sync_copy(qkv_hbm.at[pl.ds(s0, _CW), h], rows_v)
      pltpu.sync_copy(rank_hbm.at[h, pl.ds(n * S + s0, _CW)], idx_v.at[0])
      def add_base(k, _):
        idx_v[0, pl.ds(k * 16, 16)] = idx_v[0, pl.ds(k * 16, 16)] + h * NS
        return 0
      lax.fori_loop(0, _CW // 16, add_base, 0)
      pltpu.async_copy(rows_v, qkvg_hbm.at[idx_v.at[0]], sem).wait()
      return 0
    lax.fori_loop(0, S // _CW, window, 0)


def _stage_c(qkv, rank):
  mesh = plsc.VectorSubcoreMesh(core_axis_name="c", subcore_axis_name="s")
  f = pl.kernel(
      _stage_c_body,
      out_type=jax.ShapeDtypeStruct((H * NS, 2 * DH), jnp.float32),
      mesh=mesh,
      scratch_types=[
          pltpu.VMEM((_CW, 2 * DH), jnp.float32),
          pltpu.VMEM((1, _CW), jnp.int32),
          pltpu.SemaphoreType.DMA,
      ],
      compiler_params=pltpu.CompilerParams(needs_layout_passes=False),
  )
  return f(qkv, rank)


_CB = 8
_NSTEP = NC // _CB


_R = _CB * CHUNK
_K = _R + CHUNK


def _stage_d_body(qkv_ref, prev_ref, pos_ref, pprev_ref, posq_ref, band_ref,
                  out_ref):
  q = qkv_ref[:, 0:DH]
  v = qkv_ref[:, DH:2 * DH]
  kcat = jnp.concatenate([prev_ref[:, 0:DH], q], axis=0)
  vcat = jnp.concatenate([prev_ref[:, DH:2 * DH], v], axis=0)
  norm = lax.rsqrt(jnp.mean(kcat * kcat, axis=1, keepdims=True) + 1e-6)
  key = kcat * (norm * (DH ** -0.5))
  dots = lax.dot_general(q.astype(jnp.bfloat16), key.astype(jnp.bfloat16),
                         (((1,), (1,)), ((), ())),
                         preferred_element_type=jnp.float32)
  qi = posq_ref[...]
  ki = jnp.concatenate([pprev_ref[0, 0:1, :]]
                       + [pos_ref[i, 0:1, :] for i in range(_CB)],
                       axis=1)
  dots = jnp.where(qi == ki, jnp.float32(-1e5), dots)
  dots = jnp.where(qi >= ki, dots, jnp.float32(-1e9))
  dots = dots + band_ref[...]
  mx = jnp.max(dots, axis=1, keepdims=True)
  ex = jnp.exp(dots - mx)
  sm = jnp.sum(ex, axis=1, keepdims=True)
  out = lax.dot_general(ex, vcat, (((1,), (0,)), ((), ())),
                        preferred_element_type=jnp.float32) / sm
  out_ref[:, 0:DH] = out
  out_ref[:, DH:DH + 1] = mx + jnp.log(sm)


def _prev_index(i):
  h = i // _NSTEP
  j = i % _NSTEP
  return h * NC + (j * _CB + NC - 1) % NC


def _band_bias():
  ci = np.arange(_R)[:, None] // CHUNK
  jb = np.arange(_K)[None, :] // CHUNK
  band = (jb == ci) | (jb == ci + 1)
  return jnp.asarray(np.where(band, 0.0, -1e9).astype(np.float32))


def _stage_d(qkv_g, pos_g):
  qkv_c = qkv_g.reshape(H * NC, CHUNK, 2 * DH)
  pos_c = pos_g.reshape(H * NC, 1, CHUNK)
  pos_q = pos_g.reshape(H * NS, 1)
  return pl.pallas_call(
      _stage_d_body,
      grid=(H * _NSTEP,),
      in_specs=[
          pl.BlockSpec((_R, 2 * DH), lambda i: (i, 0)),
          pl.BlockSpec((CHUNK, 2 * DH), lambda i: (_prev_index(i), 0)),
          pl.BlockSpec((_CB, 1, CHUNK), lambda i: (i, 0, 0)),
          pl.BlockSpec((1, 1, CHUNK), lambda i: (_prev_index(i), 0, 0)),
          pl.BlockSpec((_R, 1), lambda i: (i, 0)),
          pl.BlockSpec((_R, _K), lambda i: (0, 0)),
      ],
      out_specs=pl.BlockSpec((_R, 2 * DH), lambda i: (i, 0)),
      out_shape=jax.ShapeDtypeStruct((H * NS, 2 * DH), jnp.float32),
      compiler_params=pltpu.CompilerParams(
          dimension_semantics=("arbitrary",),
      ),
  )(qkv_g, qkv_g, pos_c, pos_c, pos_q, _band_bias())


_EW = 64
_SHALF = S // 2


def _stage_e_body(outg_hbm, rank_hbm, final_hbm, rank_v, idx_v, rows_v,
                  w_v, acc_v, sem):
  wid = lax.axis_index("s") * 2 + lax.axis_index("c")
  h = wid // 2
  half = wid % 2
  sbase = half * _SHALF

  for n in range(NH):
    pltpu.sync_copy(rank_hbm.at[h, pl.ds(n * S + sbase, _SHALF)],
                    rank_v.at[n])

  def window(w, _):
    s0 = w * _EW
    for n in range(NH):
      def mk_idx(k, _, n=n):
        idx_v[n, pl.ds(k * 16, 16)] = (
            rank_v[n, pl.ds(s0 + k * 16, 16)] + h * NS)
        return 0
      lax.fori_loop(0, _EW // 16, mk_idx, 0)
      pltpu.async_copy(outg_hbm.at[idx_v.at[n]], rows_v.at[n], sem).wait()

    def sgroup(g, _):
      svec = g * 16 + lax.iota(jnp.int32, 16)
      c64 = jnp.full((16,), DH, jnp.int32)
      ls = [plsc.load_gather(rows_v, [jnp.full((16,), n, jnp.int32), svec, c64])
            for n in range(NH)]
      m = jnp.maximum(jnp.maximum(ls[0], ls[1]), jnp.maximum(ls[2], ls[3]))
      es = [jnp.exp(l - m) for l in ls]
      den = es[0] + es[1] + es[2] + es[3]
      for n in range(NH):
        w_v[n, pl.ds(g * 16, 16)] = es[n] / den
      return 0
    lax.fori_loop(0, _EW // 16, sgroup, 0)

    def scomb(sl, _):
      for dv in range(DH // 16):
        acc = jnp.zeros((16,), jnp.float32)
        for n in range(NH):
          wspl = plsc.load_gather(
              w_v, [jnp.full((16,), n, jnp.int32),
                    jnp.full((16,), sl, jnp.int32)])
          acc = acc + wspl * rows_v[n, sl, pl.ds(dv * 16, 16)]
        acc_v[sl, pl.ds(dv * 16, 16)] = acc
      return 0
    lax.fori_loop(0, _EW, scomb, 0)

    pltpu.sync_copy(acc_v, final_hbm.at[h, pl.ds(sbase + s0, _EW), :])
    return 0

  lax.fori_loop(0, _SHALF // _EW, window, 0)


def _stage_e(out_g, rank):
  mesh = plsc.VectorSubcoreMesh(core_axis_name="c", subcore_axis_name="s")
  f = pl.kernel(
      _stage_e_body,
      out_type=jax.ShapeDtypeStruct((H, S, DH), jnp.float32),
      mesh=mesh,
      scratch_types=[
          pltpu.VMEM((NH, _SHALF), jnp.int32),
          pltpu.VMEM((NH, _EW), jnp.int32),
          pltpu.VMEM((NH, _EW, 2 * DH), jnp.float32),
          pltpu.VMEM((NH, _EW), jnp.float32),
          pltpu.VMEM((_EW, DH), jnp.float32),
          pltpu.SemaphoreType.DMA,
      ],
      compiler_params=pltpu.CompilerParams(needs_layout_passes=False),
  )
  return f(out_g, rank)


def kernel(hidden_states, W_qk, W_v, rotations):
  hs = hidden_states[0]
  rot2 = rotations.reshape(DH, NH * (NB // 2))
  qkv, qkt = _stage_a1(hs, W_qk, W_v)
  buckets = _stage_a2(qkt, rot2)
  rank, pos_g = _stage_b(buckets)
  qkv_g = _stage_c(qkv, rank)
  out_g = _stage_d(qkv_g, pos_g)
  final = _stage_e(out_g, rank)
  return final.transpose(1, 0, 2).reshape(1, S, H * DH)


# speedup vs baseline: 1.2407x; 1.2407x over previous
import functools

import jax
import jax.numpy as jnp
import numpy as np
from jax import lax
from jax.experimental import pallas as pl
from jax.experimental.pallas import tpu as pltpu
from jax.experimental.pallas import tpu_sc as plsc

B, S, D = 1, 8192, 1024
H, DH = 16, 64
NH = 4
NB = 256
NKEY = NH * NB
CHUNK = 64
NS = NH * S
NC = NS // CHUNK


_SB = 512
_HB = 2048


def _stage_a1_body(hs_ref, wqk_ref, wv_ref, qkv_ref, qkt_ref):
  hs = hs_ref[...]
  qk = lax.dot_general(hs, wqk_ref[...], (((1,), (1,)), ((), ())),
                       preferred_element_type=jnp.float32)
  v = lax.dot_general(hs, wv_ref[...], (((1,), (1,)), ((), ())),
                      preferred_element_type=jnp.float32)
  qkt_ref[...] = qk.T
  for h in range(H):
    qkv_ref[:, h, 0:DH] = qk[:, h * DH:(h + 1) * DH]
    qkv_ref[:, h, DH:2 * DH] = v[:, h * DH:(h + 1) * DH]


def _stage_a1(hs, w_qk, w_v):
  return pl.pallas_call(
      _stage_a1_body,
      grid=(S // _SB,),
      in_specs=[
          pl.BlockSpec((_SB, D), lambda i: (i, 0)),
          pl.BlockSpec((H * DH, D), lambda i: (0, 0)),
          pl.BlockSpec((H * DH, D), lambda i: (0, 0)),
      ],
      out_specs=[
          pl.BlockSpec((_SB, H, 2 * DH), lambda i: (i, 0, 0)),
          pl.BlockSpec((H * DH, _SB), lambda i: (0, i)),
      ],
      out_shape=[
          jax.ShapeDtypeStruct((S, H, 2 * DH), jnp.float32),
          jax.ShapeDtypeStruct((H * DH, S), jnp.float32),
      ],
      compiler_params=pltpu.CompilerParams(
          dimension_semantics=("arbitrary",),
      ),
  )(hs, w_qk, w_v)


def _stage_a2_body(qkt_ref, rot_ref, bkt_ref):
  rt = lax.dot_general(rot_ref[...], qkt_ref[...], (((0,), (0,)), ((), ())),
                       preferred_element_type=jnp.float32)
  sub = lax.broadcasted_iota(jnp.int32, (NB // 2, _HB), 0)
  big = jnp.int32(1 << 20)
  for n in range(NH):
    rn = rt[n * (NB // 2):(n + 1) * (NB // 2), :]
    mx = jnp.max(rn, axis=0, keepdims=True)
    mn = jnp.min(rn, axis=0, keepdims=True)
    amx = jnp.min(jnp.where(rn == mx, sub, big), axis=0, keepdims=True)
    amn = jnp.min(jnp.where(rn == mn, sub, big), axis=0, keepdims=True)
    b = jnp.where(mx >= -mn, amx, NB // 2 + amn) + n * NB
    bkt_ref[0, n:n + 1, :] = b


def _stage_a2(qkt, rot2):
  return pl.pallas_call(
      _stage_a2_body,
      grid=(H, S // _HB),
      in_specs=[
          pl.BlockSpec((DH, _HB), lambda h, j: (h, j)),
          pl.BlockSpec((DH, NH * (NB // 2)), lambda h, j: (0, 0)),
      ],
      out_specs=pl.BlockSpec((1, NH, _HB), lambda h, j: (h, 0, j)),
      out_shape=jax.ShapeDtypeStruct((H, NH, S), jnp.int32),
      compiler_params=pltpu.CompilerParams(
          dimension_semantics=("arbitrary", "arbitrary"),
      ),
  )(qkt, rot2)


_NVEC = NS // 16


def _stage_b_body(bkt_hbm, rank_hbm, pos_hbm, bkt_v, hist_v, rank_v, pos_v):
  wid = lax.axis_index("s") * 2 + lax.axis_index("c")

  @pl.when(wid < H)
  def _():
    pltpu.sync_copy(bkt_hbm.at[wid], bkt_v)

    def zero_hist(i, _):
      hist_v[pl.ds(i * 16, 16)] = jnp.zeros((16,), jnp.int32)
      return 0
    lax.fori_loop(0, NKEY // 16, zero_hist, 0)

    def hist_pass(k, _):
      b = bkt_v[pl.ds(k * 16, 16)]
      cnt, last = plsc.scan_count(b)
      base = plsc.load_gather(hist_v, [b])
      plsc.store_scatter(hist_v, [b], base + cnt, mask=last)
      return 0
    lax.fori_loop(0, _NVEC, hist_pass, 0)

    def excl_scan(i, carry):
      v = hist_v[pl.ds(i * 16, 16)]
      c = plsc.cumsum(v)
      hist_v[pl.ds(i * 16, 16)] = carry + c - v
      return carry + jnp.sum(v)
    lax.fori_loop(0, NKEY // 16, excl_scan, jnp.int32(0))

    def rank_pass(k, _):
      b = bkt_v[pl.ds(k * 16, 16)]
      cnt, last = plsc.scan_count(b)
      base = plsc.load_gather(hist_v, [b])
      rank_v[pl.ds(k * 16, 16)] = base + cnt - 1
      plsc.store_scatter(hist_v, [b], base + cnt, mask=last)
      return 0
    lax.fori_loop(0, _NVEC, rank_pass, 0)

    pltpu.sync_copy(rank_v, rank_hbm.at[wid])

    def pos_pass(k, _):
      r = rank_v[pl.ds(k * 16, 16)]
      i_vec = k * 16 + lax.iota(jnp.int32, 16)
      plsc.store_scatter(pos_v, [r], i_vec & (S - 1))
      return 0
    lax.fori_loop(0, _NVEC, pos_pass, 0)
    pltpu.sync_copy(pos_v, pos_hbm.at[wid])


def _stage_b(buckets):
  mesh = plsc.VectorSubcoreMesh(core_axis_name="c", subcore_axis_name="s")
  f = pl.kernel(
      _stage_b_body,
      out_type=[jax.ShapeDtypeStruct((H, NS), jnp.int32),
                jax.ShapeDtypeStruct((H, NS), jnp.int32)],
      mesh=mesh,
      scratch_types=[
          pltpu.VMEM((NS,), jnp.int32),
          pltpu.VMEM((NKEY,), jnp.int32),
          pltpu.VMEM((NS,), jnp.int32),
          pltpu.VMEM((NS,), jnp.int32),
      ],
      compiler_params=pltpu.CompilerParams(needs_layout_passes=False),
  )
  return f(buckets.reshape(H, NS))


_CW = 128


def _stage_c_body(hbase, qkv_hbm, rank_hbm, qkvg_hbm, rows_v, idx_v, sem):
  wid = lax.axis_index("s") * 2 + lax.axis_index("c")
  hl = wid // 8
  rem = wid % 8
  n = rem // 2
  half = rem % 2
  h = hbase + hl
  sbase = half * (S // 2)

  def window(w, _):
    s0 = sbase + w * _CW
    pltpu.sync_copy(qkv_hbm.at[pl.ds(s0, _CW), h], rows_v)
    pltpu.sync_copy(rank_hbm.at[h, pl.ds(n * S + s0, _CW)], idx_v.at[0])
    def add_base(k, _):
      idx_v[0, pl.ds(k * 16, 16)] = idx_v[0, pl.ds(k * 16, 16)] + hl * NS
      return 0
    lax.fori_loop(0, _CW // 16, add_base, 0)
    pltpu.async_copy(rows_v, qkvg_hbm.at[idx_v.at[0]], sem).wait()
    return 0
  lax.fori_loop(0, S // 2 // _CW, window, 0)


def _stage_c(qkv, rank, hbase, gh):
  mesh = plsc.VectorSubcoreMesh(core_axis_name="c", subcore_axis_name="s")
  f = pl.kernel(
      functools.partial(_stage_c_body, hbase),
      out_type=jax.ShapeDtypeStruct((gh * NS, 2 * DH), jnp.float32),
      mesh=mesh,
      scratch_types=[
          pltpu.VMEM((_CW, 2 * DH), jnp.float32),
          pltpu.VMEM((1, _CW), jnp.int32),
          pltpu.SemaphoreType.DMA,
      ],
      compiler_params=pltpu.CompilerParams(needs_layout_passes=False),
  )
  return f(qkv, rank)


_CB = 8
_NSTEP = NC // _CB


_R = _CB * CHUNK
_K = _R + CHUNK


def _stage_d_body(qkv_ref, prev_ref, pos_ref, pprev_ref, posq_ref, out_ref):
  q = qkv_ref[:, 0:DH]
  v = qkv_ref[:, DH:2 * DH]
  kcat = jnp.concatenate([prev_ref[:, 0:DH], q], axis=0)
  vcat = jnp.concatenate([prev_ref[:, DH:2 * DH], v], axis=0)
  norm = lax.rsqrt(jnp.mean(kcat * kcat, axis=1, keepdims=True) + 1e-6)
  key = kcat * (norm * (DH ** -0.5))
  dots = lax.dot_general(q.astype(jnp.bfloat16), key.astype(jnp.bfloat16),
                         (((1,), (1,)), ((), ())),
                         preferred_element_type=jnp.float32)
  qi = posq_ref[...]
  ki = jnp.concatenate([pprev_ref[0, 0:1, :]]
                       + [pos_ref[i, 0:1, :] for i in range(_CB)],
                       axis=1)
  ci = lax.broadcasted_iota(jnp.int32, (_R, _K), 0) // CHUNK
  jb = lax.broadcasted_iota(jnp.int32, (_R, _K), 1) // CHUNK
  in_band = (jb == ci) | (jb == ci + 1)
  dots = jnp.where(qi == ki, jnp.float32(-1e5), dots)
  dots = jnp.where(in_band & (qi >= ki), dots, jnp.float32(-1e9))
  mx = jnp.max(dots, axis=1, keepdims=True)
  ex = jnp.exp(dots - mx)
  sm = jnp.sum(ex, axis=1, keepdims=True)
  out = lax.dot_general(ex, vcat, (((1,), (0,)), ((), ())),
                        preferred_element_type=jnp.float32) / sm
  out_ref[:, 0:DH] = out
  out_ref[:, DH:DH + 1] = mx + jnp.log(sm)


def _prev_index(i):
  h = i // _NSTEP
  j = i % _NSTEP
  return h * NC + (j * _CB + NC - 1) % NC


def _stage_d(qkv_g, pos_g, hbase, gh):
  pos_c = pos_g.reshape(H * NC, 1, CHUNK)
  pos_q = pos_g.reshape(H * NS, 1)
  nstep = NC // _CB

  def blk(i):
    return hbase * nstep + i

  def lprev(i):
    return (i // nstep) * NC + ((i % nstep) * _CB + NC - 1) % NC

  def gprev(i):
    return hbase * NC + lprev(i)

  return pl.pallas_call(
      _stage_d_body,
      grid=(gh * nstep,),
      in_specs=[
          pl.BlockSpec((_R, 2 * DH), lambda i: (i, 0)),
          pl.BlockSpec((CHUNK, 2 * DH), lambda i: (lprev(i), 0)),
          pl.BlockSpec((_CB, 1, CHUNK), lambda i: (blk(i), 0, 0)),
          pl.BlockSpec((1, 1, CHUNK), lambda i: (gprev(i), 0, 0)),
          pl.BlockSpec((_R, 1), lambda i: (blk(i), 0)),
      ],
      out_specs=pl.BlockSpec((_R, 2 * DH), lambda i: (i, 0)),
      out_shape=jax.ShapeDtypeStruct((gh * NS, 2 * DH), jnp.float32),
      compiler_params=pltpu.CompilerParams(
          dimension_semantics=("arbitrary",),
      ),
  )(qkv_g, qkv_g, pos_c, pos_c, pos_q)


_EW = 64
_SOCT = S // 8


def _stage_e_body(hbase, outg_hbm, rank_hbm, final_hbm, rank_v, idx_v, rows_v,
                  w_v, acc_v, sem):
  wid = lax.axis_index("s") * 2 + lax.axis_index("c")
  hl = wid // 8
  h = hbase + hl
  sbase = (wid % 8) * _SOCT

  for n in range(NH):
    pltpu.sync_copy(rank_hbm.at[h, pl.ds(n * S + sbase, _SOCT)],
                    rank_v.at[n])

  def window(w, _):
    s0 = w * _EW
    for n in range(NH):
      def mk_idx(k, _, n=n):
        idx_v[n, pl.ds(k * 16, 16)] = (
            rank_v[n, pl.ds(s0 + k * 16, 16)] + hl * NS)
        return 0
      lax.fori_loop(0, _EW // 16, mk_idx, 0)
      pltpu.async_copy(outg_hbm.at[idx_v.at[n]], rows_v.at[n], sem).wait()

    def sgroup(g, _):
      svec = g * 16 + lax.iota(jnp.int32, 16)
      c64 = jnp.full((16,), DH, jnp.int32)
      ls = [plsc.load_gather(rows_v, [jnp.full((16,), n, jnp.int32), svec, c64])
            for n in range(NH)]
      m = jnp.maximum(jnp.maximum(ls[0], ls[1]), jnp.maximum(ls[2], ls[3]))
      es = [jnp.exp(l - m) for l in ls]
      den = es[0] + es[1] + es[2] + es[3]
      for n in range(NH):
        w_v[n, pl.ds(g * 16, 16)] = es[n] / den
      return 0
    lax.fori_loop(0, _EW // 16, sgroup, 0)

    def scomb(sl, _):
      for dv in range(DH // 16):
        acc = jnp.zeros((16,), jnp.float32)
        for n in range(NH):
          wspl = plsc.load_gather(
              w_v, [jnp.full((16,), n, jnp.int32),
                    jnp.full((16,), sl, jnp.int32)])
          acc = acc + wspl * rows_v[n, sl, pl.ds(dv * 16, 16)]
        acc_v[sl, pl.ds(dv * 16, 16)] = acc
      return 0
    lax.fori_loop(0, _EW, scomb, 0)

    pltpu.sync_copy(acc_v, final_hbm.at[hl, pl.ds(sbase + s0, _EW), :])
    return 0

  lax.fori_loop(0, _SOCT // _EW, window, 0)


def _stage_e(out_g, rank, hbase, gh):
  mesh = plsc.VectorSubcoreMesh(core_axis_name="c", subcore_axis_name="s")
  f = pl.kernel(
      functools.partial(_stage_e_body, hbase),
      out_type=jax.ShapeDtypeStruct((gh, S, DH), jnp.float32),
      mesh=mesh,
      scratch_types=[
          pltpu.VMEM((NH, _SOCT), jnp.int32),
          pltpu.VMEM((NH, _EW), jnp.int32),
          pltpu.VMEM((NH, _EW, 2 * DH), jnp.float32),
          pltpu.VMEM((NH, _EW), jnp.float32),
          pltpu.VMEM((_EW, DH), jnp.float32),
          pltpu.SemaphoreType.DMA,
      ],
      compiler_params=pltpu.CompilerParams(needs_layout_passes=False),
  )
  return f(out_g, rank)


_GH = 4


def kernel(hidden_states, W_qk, W_v, rotations):
  hs = hidden_states[0]
  rot2 = rotations.reshape(DH, NH * (NB // 2))
  qkv, qkt = _stage_a1(hs, W_qk, W_v)
  buckets = _stage_a2(qkt, rot2)
  rank, pos_g = _stage_b(buckets)
  finals = []
  for g in range(H // _GH):
    hbase = g * _GH
    qkv_g = _stage_c(qkv, rank, hbase, _GH)
    out_g = _stage_d(qkv_g, pos_g, hbase, _GH)
    finals.append(_stage_e(out_g, rank, hbase, _GH))
  final = jnp.concatenate(finals, axis=0)
  return final.transpose(1, 0, 2).reshape(1, S, H * DH)


# speedup vs baseline: 1.2546x; 1.0112x over previous
import functools

import jax
import jax.numpy as jnp
import numpy as np
from jax import lax
from jax.experimental import pallas as pl
from jax.experimental.pallas import tpu as pltpu
from jax.experimental.pallas import tpu_sc as plsc

B, S, D = 1, 8192, 1024
H, DH = 16, 64
NH = 4
NB = 256
NKEY = NH * NB
CHUNK = 64
NS = NH * S
NC = NS // CHUNK


_SB = 512
_HB = 2048


def _stage_a1_body(hs_ref, wqk_ref, wv_ref, qkv_ref, qkt_ref):
  hs = hs_ref[...]
  qk = lax.dot_general(hs, wqk_ref[...], (((1,), (1,)), ((), ())),
                       preferred_element_type=jnp.float32)
  v = lax.dot_general(hs, wv_ref[...], (((1,), (1,)), ((), ())),
                      preferred_element_type=jnp.float32)
  qkt_ref[...] = qk.T
  for h in range(H):
    qkv_ref[:, h, 0:DH] = qk[:, h * DH:(h + 1) * DH]
    qkv_ref[:, h, DH:2 * DH] = v[:, h * DH:(h + 1) * DH]


def _stage_a1(hs, w_qk, w_v):
  return pl.pallas_call(
      _stage_a1_body,
      grid=(S // _SB,),
      in_specs=[
          pl.BlockSpec((_SB, D), lambda i: (i, 0)),
          pl.BlockSpec((H * DH, D), lambda i: (0, 0)),
          pl.BlockSpec((H * DH, D), lambda i: (0, 0)),
      ],
      out_specs=[
          pl.BlockSpec((_SB, H, 2 * DH), lambda i: (i, 0, 0)),
          pl.BlockSpec((H * DH, _SB), lambda i: (0, i)),
      ],
      out_shape=[
          jax.ShapeDtypeStruct((S, H, 2 * DH), jnp.float32),
          jax.ShapeDtypeStruct((H * DH, S), jnp.float32),
      ],
      compiler_params=pltpu.CompilerParams(
          dimension_semantics=("arbitrary",),
      ),
  )(hs, w_qk, w_v)


def _stage_a2_body(qkt_ref, rot_ref, bkt_ref):
  rt = lax.dot_general(rot_ref[...], qkt_ref[...], (((0,), (0,)), ((), ())),
                       preferred_element_type=jnp.float32)
  sub = lax.broadcasted_iota(jnp.int32, (NB // 2, _HB), 0)
  big = jnp.int32(1 << 20)
  for n in range(NH):
    rn = rt[n * (NB // 2):(n + 1) * (NB // 2), :]
    mx = jnp.max(rn, axis=0, keepdims=True)
    mn = jnp.min(rn, axis=0, keepdims=True)
    amx = jnp.min(jnp.where(rn == mx, sub, big), axis=0, keepdims=True)
    amn = jnp.min(jnp.where(rn == mn, sub, big), axis=0, keepdims=True)
    b = jnp.where(mx >= -mn, amx, NB // 2 + amn) + n * NB
    bkt_ref[0, n:n + 1, :] = b


def _stage_a2(qkt, rot2):
  return pl.pallas_call(
      _stage_a2_body,
      grid=(H, S // _HB),
      in_specs=[
          pl.BlockSpec((DH, _HB), lambda h, j: (h, j)),
          pl.BlockSpec((DH, NH * (NB // 2)), lambda h, j: (0, 0)),
      ],
      out_specs=pl.BlockSpec((1, NH, _HB), lambda h, j: (h, 0, j)),
      out_shape=jax.ShapeDtypeStruct((H, NH, S), jnp.int32),
      compiler_params=pltpu.CompilerParams(
          dimension_semantics=("arbitrary", "arbitrary"),
      ),
  )(qkt, rot2)


_NVEC = NS // 16


def _stage_b_body(bkt_hbm, rank_hbm, pos_hbm, bkt_v, hist_v, rank_v, pos_v):
  wid = lax.axis_index("s") * 2 + lax.axis_index("c")

  @pl.when(wid < H)
  def _():
    pltpu.sync_copy(bkt_hbm.at[wid], bkt_v)

    def zero_hist(i, _):
      hist_v[pl.ds(i * 16, 16)] = jnp.zeros((16,), jnp.int32)
      return 0
    lax.fori_loop(0, NKEY // 16, zero_hist, 0)

    def hist_pass(k, _):
      b = bkt_v[pl.ds(k * 16, 16)]
      cnt, last = plsc.scan_count(b)
      base = plsc.load_gather(hist_v, [b])
      plsc.store_scatter(hist_v, [b], base + cnt, mask=last)
      return 0
    lax.fori_loop(0, _NVEC, hist_pass, 0)

    def excl_scan(i, carry):
      v = hist_v[pl.ds(i * 16, 16)]
      c = plsc.cumsum(v)
      hist_v[pl.ds(i * 16, 16)] = carry + c - v
      return carry + jnp.sum(v)
    lax.fori_loop(0, NKEY // 16, excl_scan, jnp.int32(0))

    def rank_pass(k, _):
      b = bkt_v[pl.ds(k * 16, 16)]
      cnt, last = plsc.scan_count(b)
      base = plsc.load_gather(hist_v, [b])
      rank_v[pl.ds(k * 16, 16)] = base + cnt - 1
      plsc.store_scatter(hist_v, [b], base + cnt, mask=last)
      return 0
    lax.fori_loop(0, _NVEC, rank_pass, 0)

    pltpu.sync_copy(rank_v, rank_hbm.at[wid])

    def pos_pass(k, _):
      r = rank_v[pl.ds(k * 16, 16)]
      i_vec = k * 16 + lax.iota(jnp.int32, 16)
      plsc.store_scatter(pos_v, [r], i_vec & (S - 1))
      return 0
    lax.fori_loop(0, _NVEC, pos_pass, 0)
    pltpu.sync_copy(pos_v, pos_hbm.at[wid])


def _stage_b(buckets):
  mesh = plsc.VectorSubcoreMesh(core_axis_name="c", subcore_axis_name="s")
  f = pl.kernel(
      _stage_b_body,
      out_type=[jax.ShapeDtypeStruct((H, NS), jnp.int32),
                jax.ShapeDtypeStruct((H, NS), jnp.int32)],
      mesh=mesh,
      scratch_types=[
          pltpu.VMEM((NS,), jnp.int32),
          pltpu.VMEM((NKEY,), jnp.int32),
          pltpu.VMEM((NS,), jnp.int32),
          pltpu.VMEM((NS,), jnp.int32),
      ],
      compiler_params=pltpu.CompilerParams(needs_layout_passes=False),
  )
  return f(buckets.reshape(H, NS))


_CW = 128


def _stage_c_body(hbase, qkv_hbm, rank_hbm, qkvg_hbm, rows_v, idx_v, sem):
  wid = lax.axis_index("s") * 2 + lax.axis_index("c")
  hl = wid // 8
  rem = wid % 8
  n = rem // 2
  half = rem % 2
  h = hbase + hl
  sbase = half * (S // 2)

  def window(w, _):
    s0 = sbase + w * _CW
    pltpu.sync_copy(qkv_hbm.at[pl.ds(s0, _CW), h], rows_v)
    pltpu.sync_copy(rank_hbm.at[h, pl.ds(n * S + s0, _CW)], idx_v.at[0])
    def add_base(k, _):
      idx_v[0, pl.ds(k * 16, 16)] = idx_v[0, pl.ds(k * 16, 16)] + hl * NS
      return 0
    lax.fori_loop(0, _CW // 16, add_base, 0)
    pltpu.async_copy(rows_v, qkvg_hbm.at[idx_v.at[0]], sem).wait()
    return 0
  lax.fori_loop(0, S // 2 // _CW, window, 0)


def _stage_c(qkv, rank, hbase, gh):
  mesh = plsc.VectorSubcoreMesh(core_axis_name="c", subcore_axis_name="s")
  f = pl.kernel(
      functools.partial(_stage_c_body, hbase),
      out_type=jax.ShapeDtypeStruct((gh * NS, 2 * DH), jnp.float32),
      mesh=mesh,
      scratch_types=[
          pltpu.VMEM((_CW, 2 * DH), jnp.float32),
          pltpu.VMEM((1, _CW), jnp.int32),
          pltpu.SemaphoreType.DMA,
      ],
      compiler_params=pltpu.CompilerParams(needs_layout_passes=False),
  )
  return f(qkv, rank)


_CB = 8
_NSTEP = NC // _CB


_R = _CB * CHUNK
_K = _R + CHUNK


def _stage_d_body(qkv_ref, prev_ref, pos_ref, pprev_ref, posq_ref, out_ref):
  q = qkv_ref[:, 0:DH]
  v = qkv_ref[:, DH:2 * DH]
  kcat = jnp.concatenate([prev_ref[:, 0:DH], q], axis=0)
  vcat = jnp.concatenate([prev_ref[:, DH:2 * DH], v], axis=0)
  norm = lax.rsqrt(jnp.mean(kcat * kcat, axis=1, keepdims=True) + 1e-6)
  key = kcat * (norm * (DH ** -0.5))
  dots = lax.dot_general(q.astype(jnp.bfloat16), key.astype(jnp.bfloat16),
                         (((1,), (1,)), ((), ())),
                         preferred_element_type=jnp.float32)
  qi = posq_ref[...]
  ki = jnp.concatenate([pprev_ref[0, 0:1, :]]
                       + [pos_ref[i, 0:1, :] for i in range(_CB)],
                       axis=1)
  ci = lax.broadcasted_iota(jnp.int32, (_R, _K), 0) // CHUNK
  jb = lax.broadcasted_iota(jnp.int32, (_R, _K), 1) // CHUNK
  in_band = (jb == ci) | (jb == ci + 1)
  dots = jnp.where(qi == ki, jnp.float32(-1e5), dots)
  dots = jnp.where(in_band & (qi >= ki), dots, jnp.float32(-1e9))
  mx = jnp.max(dots, axis=1, keepdims=True)
  ex = jnp.exp(dots - mx)
  sm = jnp.sum(ex, axis=1, keepdims=True)
  out = lax.dot_general(ex, vcat, (((1,), (0,)), ((), ())),
                        preferred_element_type=jnp.float32) / sm
  out_ref[:, 0:DH] = out
  out_ref[:, DH:DH + 1] = mx + jnp.log(sm)


def _prev_index(i):
  h = i // _NSTEP
  j = i % _NSTEP
  return h * NC + (j * _CB + NC - 1) % NC


def _stage_d(qkv_g, pos_g, hbase, gh):
  pos_c = pos_g.reshape(H * NC, 1, CHUNK)
  pos_q = pos_g.reshape(H * NS, 1)
  nstep = NC // _CB

  def blk(i):
    return hbase * nstep + i

  def lprev(i):
    return (i // nstep) * NC + ((i % nstep) * _CB + NC - 1) % NC

  def gprev(i):
    return hbase * NC + lprev(i)

  return pl.pallas_call(
      _stage_d_body,
      grid=(gh * nstep,),
      in_specs=[
          pl.BlockSpec((_R, 2 * DH), lambda i: (i, 0)),
          pl.BlockSpec((CHUNK, 2 * DH), lambda i: (lprev(i), 0)),
          pl.BlockSpec((_CB, 1, CHUNK), lambda i: (blk(i), 0, 0)),
          pl.BlockSpec((1, 1, CHUNK), lambda i: (gprev(i), 0, 0)),
          pl.BlockSpec((_R, 1), lambda i: (blk(i), 0)),
      ],
      out_specs=pl.BlockSpec((_R, 2 * DH), lambda i: (i, 0)),
      out_shape=jax.ShapeDtypeStruct((gh * NS, 2 * DH), jnp.float32),
      compiler_params=pltpu.CompilerParams(
          dimension_semantics=("arbitrary",),
      ),
  )(qkv_g, qkv_g, pos_c, pos_c, pos_q)


_EW = 64
_SOCT = S // 8


def _stage_e_body(hbase, outg_hbm, rank_hbm, final_hbm, rank_v, idx_v, rows_v,
                  w_v, acc_v, sem):
  wid = lax.axis_index("s") * 2 + lax.axis_index("c")
  hl = wid // 8
  h = hbase + hl
  sbase = (wid % 8) * _SOCT

  for n in range(NH):
    pltpu.sync_copy(rank_hbm.at[h, pl.ds(n * S + sbase, _SOCT)],
                    rank_v.at[n])

  def window(w, _):
    s0 = w * _EW
    copies = []
    for n in range(NH):
      def mk_idx(k, _, n=n):
        idx_v[n, pl.ds(k * 16, 16)] = (
            rank_v[n, pl.ds(s0 + k * 16, 16)] + hl * NS)
        return 0
      lax.fori_loop(0, _EW // 16, mk_idx, 0)
      copies.append(pltpu.async_copy(outg_hbm.at[idx_v.at[n]],
                                     rows_v.at[n], sem))
    for cp in copies:
      cp.wait()

    def sgroup(g, _):
      svec = g * 16 + lax.iota(jnp.int32, 16)
      c64 = jnp.full((16,), DH, jnp.int32)
      ls = [plsc.load_gather(rows_v, [jnp.full((16,), n, jnp.int32), svec, c64])
            for n in range(NH)]
      m = jnp.maximum(jnp.maximum(ls[0], ls[1]), jnp.maximum(ls[2], ls[3]))
      es = [jnp.exp(l - m) for l in ls]
      den = es[0] + es[1] + es[2] + es[3]
      for n in range(NH):
        w_v[n, pl.ds(g * 16, 16)] = es[n] / den
      return 0
    lax.fori_loop(0, _EW // 16, sgroup, 0)

    def scomb(sl, _):
      for dv in range(DH // 16):
        acc = jnp.zeros((16,), jnp.float32)
        for n in range(NH):
          wspl = plsc.load_gather(
              w_v, [jnp.full((16,), n, jnp.int32),
                    jnp.full((16,), sl, jnp.int32)])
          acc = acc + wspl * rows_v[n, sl, pl.ds(dv * 16, 16)]
        acc_v[sl, pl.ds(dv * 16, 16)] = acc
      return 0
    lax.fori_loop(0, _EW, scomb, 0)

    pltpu.sync_copy(acc_v, final_hbm.at[hl, pl.ds(sbase + s0, _EW), :])
    return 0

  lax.fori_loop(0, _SOCT // _EW, window, 0)


def _stage_e(out_g, rank, hbase, gh):
  mesh = plsc.VectorSubcoreMesh(core_axis_name="c", subcore_axis_name="s")
  f = pl.kernel(
      functools.partial(_stage_e_body, hbase),
      out_type=jax.ShapeDtypeStruct((gh, S, DH), jnp.float32),
      mesh=mesh,
      scratch_types=[
          pltpu.VMEM((NH, _SOCT), jnp.int32),
          pltpu.VMEM((NH, _EW), jnp.int32),
          pltpu.VMEM((NH, _EW, 2 * DH), jnp.float32),
          pltpu.VMEM((NH, _EW), jnp.float32),
          pltpu.VMEM((_EW, DH), jnp.float32),
          pltpu.SemaphoreType.DMA,
      ],
      compiler_params=pltpu.CompilerParams(needs_layout_passes=False),
  )
  return f(out_g, rank)


_GH = 4


def kernel(hidden_states, W_qk, W_v, rotations):
  hs = hidden_states[0]
  rot2 = rotations.reshape(DH, NH * (NB // 2))
  qkv, qkt = _stage_a1(hs, W_qk, W_v)
  buckets = _stage_a2(qkt, rot2)
  rank, pos_g = _stage_b(buckets)
  finals = []
  for g in range(H // _GH):
    hbase = g * _GH
    qkv_g = _stage_c(qkv, rank, hbase, _GH)
    out_g = _stage_d(qkv_g, pos_g, hbase, _GH)
    finals.append(_stage_e(out_g, rank, hbase, _GH))
  final = jnp.concatenate(finals, axis=0)
  return final.transpose(1, 0, 2).reshape(1, S, H * DH)
